# baseline (speedup 1.0000x reference)
import functools

import jax
import jax.numpy as jnp
from jax import lax
from jax.experimental import pallas as pl
from jax.experimental.pallas import tpu as pltpu
from jax.experimental.pallas import tpu_sc as plsc

N = 10000
E = 320000
K = 64
F = 128

_NC = 2
_NS = 16
_L = 16
_NW = _NC * _NS
_CHUNK = 80
_ROWS_PER_TILE = 624
_TAIL = N - _NS * _ROWS_PER_TILE
_GBLK = 2560
_H1 = 163840
_H2 = E - _H1


def _mm_bias_body(x_ref, w_ref, b_ref, o_ref):
    o_ref[...] = (
        jnp.dot(x_ref[...], w_ref[...], preferred_element_type=jnp.float32)
        + b_ref[...]
    )


def _mm_bias(x, w, b2d, blk):
    n = x.shape[0]
    return pl.pallas_call(
        _mm_bias_body,
        grid=(n // blk,),
        in_specs=[
            pl.BlockSpec((blk, x.shape[1]), lambda i: (i, 0)),
            pl.BlockSpec(w.shape, lambda i: (0, 0)),
            pl.BlockSpec((1, w.shape[1]), lambda i: (0, 0)),
        ],
        out_specs=pl.BlockSpec((blk, w.shape[1]), lambda i: (i, 0)),
        out_shape=jax.ShapeDtypeStruct((n, w.shape[1]), jnp.float32),
    )(x, w, b2d)


def _g_body(r_ref, w_ref, o_ref):
    o_ref[...] = jnp.dot(r_ref[...], w_ref[...], preferred_element_type=jnp.float32)


def _g_mm(rbf, wk2f, off_rows, n_rows, blk=_GBLK):
    off_blk = off_rows // blk
    return pl.pallas_call(
        _g_body,
        grid=(n_rows // blk,),
        in_specs=[
            pl.BlockSpec((blk, K), lambda i: (i + off_blk, 0)),
            pl.BlockSpec((K, F), lambda i: (0, 0)),
        ],
        out_specs=pl.BlockSpec((blk, F), lambda i: (i, 0)),
        out_shape=jax.ShapeDtypeStruct((n_rows, F), jnp.float32),
    )(rbf, wk2f)


def _epi_body(x_ref, xja_ref, xjb_ref, xjc_ref, xjd_ref, wi_ref, bi_ref,
              i0w1, i0b1, i0w2, i0b2, i1w1, i1b1, i1w2, i1b2,
              a0w1, a0b1, a0w2, a0b2, a1w1, a1b1, a1w2, a1b2,
              wd_ref, bd_ref, u_ref, o_ref):
    def mm(a, w):
        return jnp.dot(a, w[...], preferred_element_type=jnp.float32)

    def res(v, w1, b1, w2, b2):
        return v + mm(mm(v, w1) + b1[...], w2) + b2[...]

    x = x_ref[...]
    m = (mm(x, wi_ref) + bi_ref[...] + xja_ref[...] + xjb_ref[...]
         + xjc_ref[...] + xjd_ref[...])
    m = res(m, i0w1, i0b1, i0w2, i0b2)
    m = res(m, i1w1, i1b1, i1w2, i1b2)
    out = u_ref[...] * x + mm(m, wd_ref) + bd_ref[...]
    out = res(out, a0w1, a0b1, a0w2, a0b2)
    out = res(out, a1w1, a1b1, a1w2, a1b2)
    o_ref[...] = out


def _epilogue(x, xjA, xjB, wi, bi2, rws, wd, bd2, u2, blk=2000):
    ng = N // blk
    row_spec = pl.BlockSpec((blk, F), lambda i: (i, 0))
    xjb_spec = pl.BlockSpec((blk, F), lambda i: (i + ng, 0))
    w_spec = pl.BlockSpec((F, F), lambda i: (0, 0))
    b_spec = pl.BlockSpec((1, F), lambda i: (0, 0))
    rw_specs = []
    for _ in range(4):
        rw_specs += [w_spec, b_spec, w_spec, b_spec]
    return pl.pallas_call(
        _epi_body,
        grid=(ng,),
        in_specs=[row_spec, row_spec, xjb_spec, row_spec, xjb_spec,
                  w_spec, b_spec]
        + rw_specs + [w_spec, b_spec, b_spec],
        out_specs=row_spec,
        out_shape=jax.ShapeDtypeStruct((N, F), jnp.float32),
    )(x, xjA, xjA, xjB, xjB, wi, bi2, *rws, wd, bd2, u2)


def _sc_edge(g, xj_src, idx_i, idx_j, zrows, edge_off, cpt):
    mesh = plsc.VectorSubcoreMesh(core_axis_name="c", subcore_axis_name="s")

    @functools.partial(
        pl.kernel,
        out_type=jax.ShapeDtypeStruct((2 * N, F), jnp.float32),
        mesh=mesh,
        scratch_types=[
            pltpu.VMEM((_CHUNK,), jnp.int32),
            pltpu.VMEM((_CHUNK,), jnp.int32),
            pltpu.VMEM((_CHUNK,), jnp.int32),
            pltpu.VMEM((_CHUNK,), jnp.int32),
            pltpu.VMEM((_CHUNK, F), jnp.float32),
            pltpu.VMEM((_CHUNK, F), jnp.float32),
            pltpu.VMEM((_CHUNK, F), jnp.float32),
            pltpu.VMEM((_CHUNK, F), jnp.float32),
            pltpu.VMEM_SHARED((N, F), jnp.float32),
            pltpu.SemaphoreType.DMA,
            pltpu.SemaphoreType.DMA,
            pltpu.SemaphoreType.DMA,
            pltpu.SemaphoreType.DMA,
            pltpu.SemaphoreType.DMA,
            pltpu.SemaphoreType.DMA,
        ],
    )
    def k(g_hbm, xj_hbm, ii_hbm, ij_hbm, z_hbm, out_hbm,
          ij0, ij1, ii0, ii1, rows0, rows1, g0, g1, acc_sh,
          sl0, sl1, sg0, sg1, ss0, ss1):
        c = lax.axis_index("c")
        s = lax.axis_index("s")
        w = s * _NC + c
        bufs = ((ij0, ii0, rows0, g0, sl0, sg0, ss0),
                (ij1, ii1, rows1, g1, sl1, sg1, ss1))

        pltpu.sync_copy(
            z_hbm.at[pl.ds(0, _ROWS_PER_TILE)],
            acc_sh.at[pl.ds(s * _ROWS_PER_TILE, _ROWS_PER_TILE)],
        )

        @pl.when(s == _NS - 1)
        def _zero_tail():
            pltpu.sync_copy(
                z_hbm.at[pl.ds(0, _TAIL)],
                acc_sh.at[pl.ds(_NS * _ROWS_PER_TILE, _TAIL)],
            )

        plsc.subcore_barrier()

        def issue_linear(t, buf):
            ij_v, ii_v, _rows_v, g_v, sl, _sg, _ss = buf
            base = (w + t * _NW) * _CHUNK
            pltpu.async_copy(ij_hbm.at[pl.ds(edge_off + base, _CHUNK)], ij_v, sl)
            pltpu.async_copy(ii_hbm.at[pl.ds(edge_off + base, _CHUNK)], ii_v, sl)
            pltpu.async_copy(g_hbm.at[pl.ds(base, _CHUNK)], g_v, sl)

        def wait_linear_issue_gather(buf):
            ij_v, ii_v, rows_v, g_v, sl, sg, _ss = buf
            pltpu.make_async_copy(ij_hbm.at[pl.ds(0, _CHUNK)], ij_v, sl).wait()
            pltpu.make_async_copy(ii_hbm.at[pl.ds(0, _CHUNK)], ii_v, sl).wait()
            pltpu.make_async_copy(g_hbm.at[pl.ds(0, _CHUNK)], g_v, sl).wait()
            pltpu.async_copy(xj_hbm.at[ij_v], rows_v, sg)

        def wait_scatter(buf):
            ij_v, ii_v, rows_v, g_v, _sl, _sg, ss = buf
            pltpu.make_async_copy(g_v, acc_sh.at[ii_v], ss).wait()

        def finish(t, buf):
            ij_v, ii_v, rows_v, g_v, _sl, sg, ss = buf
            pltpu.make_async_copy(xj_hbm.at[ij_v], rows_v, sg).wait()

            @plsc.parallel_loop(0, _CHUNK * (F // _L), unroll=8)
            def _mul(t2):
                e = t2 // (F // _L)
                q = lax.rem(t2, F // _L) * _L
                g_v[e, pl.ds(q, _L)] = g_v[e, pl.ds(q, _L)] * rows_v[e, pl.ds(q, _L)]

            pltpu.async_copy(g_v, acc_sh.at[ii_v], ss, add=True)

        issue_linear(jnp.int32(0), bufs[0])
        wait_linear_issue_gather(bufs[0])

        @pl.loop(0, (cpt - 1) // 2)
        def _pair(p):
            for b in range(2):
                t = 2 * p + b
                cur, nxt = bufs[b], bufs[1 - b]

                if b == 0:
                    @pl.when(t >= 1)
                    def _drain(nxt=nxt):
                        wait_scatter(nxt)
                else:
                    wait_scatter(nxt)

                issue_linear(t + 1, nxt)
                finish(t, cur)
                wait_linear_issue_gather(nxt)

        if cpt % 2 == 0:
            wait_scatter(bufs[1])
            issue_linear(jnp.int32(cpt - 1), bufs[1])
            finish(cpt - 2, bufs[0])
            wait_linear_issue_gather(bufs[1])
            wait_scatter(bufs[0])
            finish(cpt - 1, bufs[1])
            wait_scatter(bufs[1])
        else:
            wait_scatter(bufs[1])
            finish(cpt - 1, bufs[0])
            wait_scatter(bufs[0])
        plsc.subcore_barrier()
        pltpu.sync_copy(
            acc_sh.at[pl.ds(s * _ROWS_PER_TILE, _ROWS_PER_TILE)],
            out_hbm.at[pl.ds(c * N + s * _ROWS_PER_TILE, _ROWS_PER_TILE)],
        )

        @pl.when(s == _NS - 1)
        def _out_tail():
            pltpu.sync_copy(
                acc_sh.at[pl.ds(_NS * _ROWS_PER_TILE, _TAIL)],
                out_hbm.at[pl.ds(c * N + _NS * _ROWS_PER_TILE, _TAIL)],
            )

    return k(g, xj_src, idx_i, idx_j, zrows)


def kernel(x, rbf, idx_i, idx_j, Wk2f, Wi, bi, Wj, bj,
           i0_W1, i0_b1, i0_W2, i0_b2, i1_W1, i1_b1, i1_W2, i1_b2,
           a0_W1, a0_b1, a0_W2, a0_b2, a1_W1, a1_b1, a1_W2, a1_b2,
           Wd, bd, u):
    xj_src = _mm_bias(x, Wj, bj.reshape(1, F), blk=2000)
    ii = idx_i.astype(jnp.int32)
    ij = idx_j.astype(jnp.int32)
    zrows = jnp.zeros((_ROWS_PER_TILE, F), jnp.float32)
    g1 = _g_mm(rbf, Wk2f, 0, _H1)
    xjA = _sc_edge(g1, xj_src, ii, ij, zrows, 0, _H1 // (_CHUNK * _NW))
    g2 = _g_mm(rbf, Wk2f, _H1, _H2)
    xjB = _sc_edge(g2, xj_src, ii, ij, zrows, _H1, _H2 // (_CHUNK * _NW))
    rws = (i0_W1, i0_b1.reshape(1, F), i0_W2, i0_b2.reshape(1, F),
           i1_W1, i1_b1.reshape(1, F), i1_W2, i1_b2.reshape(1, F),
           a0_W1, a0_b1.reshape(1, F), a0_W2, a0_b2.reshape(1, F),
           a1_W1, a1_b1.reshape(1, F), a1_W2, a1_b2.reshape(1, F))
    return _epilogue(x, xjA, xjB, Wi, bi.reshape(1, F), rws, Wd,
                     bd.reshape(1, F), u.reshape(1, F))


# speedup vs baseline: 1.0233x; 1.0233x over previous
import functools

import numpy as np

import jax
import jax.numpy as jnp
from jax import lax
from jax.experimental import pallas as pl
from jax.experimental.pallas import tpu as pltpu
from jax.experimental.pallas import tpu_sc as plsc

N = 10000
E = 320000
K = 64
F = 128

_NC = 2
_NS = 16
_L = 16
_NW = _NC * _NS
_CHUNK = 80
_ROWS_PER_TILE = 624
_TAIL = N - _NS * _ROWS_PER_TILE
_GBLK = 2560
_H1 = 163840
_H2 = E - _H1

_LO = np.concatenate([np.arange(32 * j, 32 * j + 16) for j in range(F // 32)])
_HI = _LO + 16


def _mm_bias_body(x_ref, w_ref, b_ref, o_ref):
    o_ref[...] = (
        jnp.dot(x_ref[...], w_ref[...], preferred_element_type=jnp.float32)
        + b_ref[...]
    )


def _mm_bias(x, w, b2d, blk):
    n = x.shape[0]
    return pl.pallas_call(
        _mm_bias_body,
        grid=(n // blk,),
        in_specs=[
            pl.BlockSpec((blk, x.shape[1]), lambda i: (i, 0)),
            pl.BlockSpec(w.shape, lambda i: (0, 0)),
            pl.BlockSpec((1, w.shape[1]), lambda i: (0, 0)),
        ],
        out_specs=pl.BlockSpec((blk, w.shape[1]), lambda i: (i, 0)),
        out_shape=jax.ShapeDtypeStruct((n, w.shape[1]), jnp.float32),
    )(x, w, b2d)


def _g_body(r_ref, wa_ref, wb_ref, o_ref):
    r = r_ref[...]
    ga = jnp.dot(r, wa_ref[...], preferred_element_type=jnp.float32)
    gb = jnp.dot(r, wb_ref[...], preferred_element_type=jnp.float32)
    ga16 = jax.lax.bitcast_convert_type(ga.astype(jnp.bfloat16), jnp.uint16)
    gb16 = jax.lax.bitcast_convert_type(gb.astype(jnp.bfloat16), jnp.uint16)
    o_ref[...] = (gb16.astype(jnp.int32) << 16) | ga16.astype(jnp.int32)


def _g_mm(rbf, wa, wb, off_rows, n_rows, blk=_GBLK):
    off_blk = off_rows // blk
    return pl.pallas_call(
        _g_body,
        grid=(n_rows // blk,),
        in_specs=[
            pl.BlockSpec((blk, K), lambda i: (i + off_blk, 0)),
            pl.BlockSpec((K, F // 2), lambda i: (0, 0)),
            pl.BlockSpec((K, F // 2), lambda i: (0, 0)),
        ],
        out_specs=pl.BlockSpec((blk, F // 2), lambda i: (i, 0)),
        out_shape=jax.ShapeDtypeStruct((n_rows, F // 2), jnp.int32),
    )(rbf, wa, wb)


def _epi_body(x_ref, xja_ref, xjb_ref, xjc_ref, xjd_ref, wi_ref, bi_ref,
              i0w1, i0b1, i0w2, i0b2, i1w1, i1b1, i1w2, i1b2,
              a0w1, a0b1, a0w2, a0b2, a1w1, a1b1, a1w2, a1b2,
              wd_ref, bd_ref, u_ref, o_ref):
    def mm(a, w):
        return jnp.dot(a, w[...], preferred_element_type=jnp.float32)

    def res(v, w1, b1, w2, b2):
        return v + mm(mm(v, w1) + b1[...], w2) + b2[...]

    x = x_ref[...]
    m = (mm(x, wi_ref) + bi_ref[...] + xja_ref[...] + xjb_ref[...]
         + xjc_ref[...] + xjd_ref[...])
    m = res(m, i0w1, i0b1, i0w2, i0b2)
    m = res(m, i1w1, i1b1, i1w2, i1b2)
    out = u_ref[...] * x + mm(m, wd_ref) + bd_ref[...]
    out = res(out, a0w1, a0b1, a0w2, a0b2)
    out = res(out, a1w1, a1b1, a1w2, a1b2)
    o_ref[...] = out


def _epilogue(x, xjA, xjB, wi, bi2, rws, wd, bd2, u2, blk=2000):
    ng = N // blk
    row_spec = pl.BlockSpec((blk, F), lambda i: (i, 0))
    xjb_spec = pl.BlockSpec((blk, F), lambda i: (i + ng, 0))
    w_spec = pl.BlockSpec((F, F), lambda i: (0, 0))
    b_spec = pl.BlockSpec((1, F), lambda i: (0, 0))
    rw_specs = []
    for _ in range(4):
        rw_specs += [w_spec, b_spec, w_spec, b_spec]
    return pl.pallas_call(
        _epi_body,
        grid=(ng,),
        in_specs=[row_spec, row_spec, xjb_spec, row_spec, xjb_spec,
                  w_spec, b_spec]
        + rw_specs + [w_spec, b_spec, b_spec],
        out_specs=row_spec,
        out_shape=jax.ShapeDtypeStruct((N, F), jnp.float32),
    )(x, xjA, xjA, xjB, xjB, wi, bi2, *rws, wd, bd2, u2)


def _sc_edge(g, xj_src, idx_i, idx_j, zrows, edge_off, cpt):
    mesh = plsc.VectorSubcoreMesh(core_axis_name="c", subcore_axis_name="s")

    @functools.partial(
        pl.kernel,
        out_type=jax.ShapeDtypeStruct((2 * N, F), jnp.float32),
        mesh=mesh,
        scratch_types=[
            pltpu.VMEM((_CHUNK,), jnp.int32),
            pltpu.VMEM((_CHUNK,), jnp.int32),
            pltpu.VMEM((_CHUNK,), jnp.int32),
            pltpu.VMEM((_CHUNK,), jnp.int32),
            pltpu.VMEM((_CHUNK, F), jnp.float32),
            pltpu.VMEM((_CHUNK, F), jnp.float32),
            pltpu.VMEM((_CHUNK, F // 2), jnp.int32),
            pltpu.VMEM((_CHUNK, F // 2), jnp.int32),
            pltpu.VMEM_SHARED((N, F), jnp.float32),
            pltpu.SemaphoreType.DMA,
            pltpu.SemaphoreType.DMA,
            pltpu.SemaphoreType.DMA,
            pltpu.SemaphoreType.DMA,
            pltpu.SemaphoreType.DMA,
            pltpu.SemaphoreType.DMA,
        ],
    )
    def k(g_hbm, xj_hbm, ii_hbm, ij_hbm, z_hbm, out_hbm,
          ij0, ij1, ii0, ii1, rows0, rows1, g0, g1, acc_sh,
          sl0, sl1, sg0, sg1, ss0, ss1):
        c = lax.axis_index("c")
        s = lax.axis_index("s")
        w = s * _NC + c
        bufs = ((ij0, ii0, rows0, g0, sl0, sg0, ss0),
                (ij1, ii1, rows1, g1, sl1, sg1, ss1))

        pltpu.sync_copy(
            z_hbm.at[pl.ds(0, _ROWS_PER_TILE)],
            acc_sh.at[pl.ds(s * _ROWS_PER_TILE, _ROWS_PER_TILE)],
        )

        @pl.when(s == _NS - 1)
        def _zero_tail():
            pltpu.sync_copy(
                z_hbm.at[pl.ds(0, _TAIL)],
                acc_sh.at[pl.ds(_NS * _ROWS_PER_TILE, _TAIL)],
            )

        plsc.subcore_barrier()

        def issue_linear(t, buf):
            ij_v, ii_v, _rows_v, g_v, sl, _sg, _ss = buf
            base = (w + t * _NW) * _CHUNK
            pltpu.async_copy(ij_hbm.at[pl.ds(edge_off + base, _CHUNK)], ij_v, sl)
            pltpu.async_copy(ii_hbm.at[pl.ds(edge_off + base, _CHUNK)], ii_v, sl)
            pltpu.async_copy(g_hbm.at[pl.ds(base, _CHUNK)], g_v, sl)

        def wait_linear_issue_gather(buf):
            ij_v, ii_v, rows_v, g_v, sl, sg, _ss = buf
            pltpu.make_async_copy(ij_hbm.at[pl.ds(0, _CHUNK)], ij_v, sl).wait()
            pltpu.make_async_copy(ii_hbm.at[pl.ds(0, _CHUNK)], ii_v, sl).wait()
            pltpu.make_async_copy(g_hbm.at[pl.ds(0, _CHUNK)], g_v, sl).wait()
            pltpu.async_copy(xj_hbm.at[ij_v], rows_v, sg)

        def wait_scatter(buf):
            ij_v, ii_v, rows_v, g_v, _sl, _sg, ss = buf
            pltpu.make_async_copy(rows_v, acc_sh.at[ii_v], ss).wait()

        def finish(t, buf):
            ij_v, ii_v, rows_v, g_v, _sl, sg, ss = buf
            pltpu.make_async_copy(xj_hbm.at[ij_v], rows_v, sg).wait()
            himask = jnp.int32(-65536)

            @plsc.parallel_loop(0, _CHUNK * (F // 32), unroll=8)
            def _mul(t2):
                e = t2 // (F // 32)
                j = lax.rem(t2, F // 32) * _L
                q = 2 * j
                gw = g_v[e, pl.ds(j, _L)]
                a = jax.lax.bitcast_convert_type(gw << 16, jnp.float32)
                b = jax.lax.bitcast_convert_type(gw & himask, jnp.float32)
                rows_v[e, pl.ds(q, _L)] = a * rows_v[e, pl.ds(q, _L)]
                rows_v[e, pl.ds(q + _L, _L)] = b * rows_v[e, pl.ds(q + _L, _L)]

            pltpu.async_copy(rows_v, acc_sh.at[ii_v], ss, add=True)

        issue_linear(jnp.int32(0), bufs[0])
        wait_linear_issue_gather(bufs[0])

        @pl.loop(0, (cpt - 1) // 2)
        def _pair(p):
            for b in range(2):
                t = 2 * p + b
                cur, nxt = bufs[b], bufs[1 - b]

                if b == 0:
                    @pl.when(t >= 1)
                    def _drain(nxt=nxt):
                        wait_scatter(nxt)
                else:
                    wait_scatter(nxt)

                issue_linear(t + 1, nxt)
                finish(t, cur)
                wait_linear_issue_gather(nxt)

        if cpt % 2 == 0:
            wait_scatter(bufs[1])
            issue_linear(jnp.int32(cpt - 1), bufs[1])
            finish(cpt - 2, bufs[0])
            wait_linear_issue_gather(bufs[1])
            wait_scatter(bufs[0])
            finish(cpt - 1, bufs[1])
            wait_scatter(bufs[1])
        else:
            wait_scatter(bufs[1])
            finish(cpt - 1, bufs[0])
            wait_scatter(bufs[0])
        plsc.subcore_barrier()
        pltpu.sync_copy(
            acc_sh.at[pl.ds(s * _ROWS_PER_TILE, _ROWS_PER_TILE)],
            out_hbm.at[pl.ds(c * N + s * _ROWS_PER_TILE, _ROWS_PER_TILE)],
        )

        @pl.when(s == _NS - 1)
        def _out_tail():
            pltpu.sync_copy(
                acc_sh.at[pl.ds(_NS * _ROWS_PER_TILE, _TAIL)],
                out_hbm.at[pl.ds(c * N + _NS * _ROWS_PER_TILE, _TAIL)],
            )

    return k(g, xj_src, idx_i, idx_j, zrows)


def kernel(x, rbf, idx_i, idx_j, Wk2f, Wi, bi, Wj, bj,
           i0_W1, i0_b1, i0_W2, i0_b2, i1_W1, i1_b1, i1_W2, i1_b2,
           a0_W1, a0_b1, a0_W2, a0_b2, a1_W1, a1_b1, a1_W2, a1_b2,
           Wd, bd, u):
    xj_src = _mm_bias(x, Wj, bj.reshape(1, F), blk=2000)
    ii = idx_i.astype(jnp.int32)
    ij = idx_j.astype(jnp.int32)
    zrows = jnp.zeros((_ROWS_PER_TILE, F), jnp.float32)
    wa = Wk2f[:, _LO]
    wb = Wk2f[:, _HI]
    g1 = _g_mm(rbf, wa, wb, 0, _H1)
    xjA = _sc_edge(g1, xj_src, ii, ij, zrows, 0, _H1 // (_CHUNK * _NW))
    g2 = _g_mm(rbf, wa, wb, _H1, _H2)
    xjB = _sc_edge(g2, xj_src, ii, ij, zrows, _H1, _H2 // (_CHUNK * _NW))
    rws = (i0_W1, i0_b1.reshape(1, F), i0_W2, i0_b2.reshape(1, F),
           i1_W1, i1_b1.reshape(1, F), i1_W2, i1_b2.reshape(1, F),
           a0_W1, a0_b1.reshape(1, F), a0_W2, a0_b2.reshape(1, F),
           a1_W1, a1_b1.reshape(1, F), a1_W2, a1_b2.reshape(1, F))
    return _epilogue(x, xjA, xjB, Wi, bi.reshape(1, F), rws, Wd,
                     bd.reshape(1, F), u.reshape(1, F))
